# Initial kernel scaffold; baseline (speedup 1.0000x reference)
import jax
import jax.numpy as jnp
from jax.experimental import pallas as pl


def kernel(x, table):
    raise NotImplementedError("write your pallas kernel here")


# speedup vs baseline: 7.4178x; 7.4178x over previous
import jax
import jax.numpy as jnp
from jax.experimental import pallas as pl
from jax.experimental.pallas import tpu as pltpu
from jax.experimental.pallas import tpu_sc as plsc

D_MODEL = 128
WINDOW = 128


def kernel(x, table):
    B, L = x.shape
    N = B * L
    idx = x.reshape(1, N)
    mesh = plsc.VectorSubcoreMesh(core_axis_name="c", subcore_axis_name="s")

    @jax.jit
    def run(table, idx):
        @pl.kernel(
            out_type=jax.ShapeDtypeStruct((N, D_MODEL), table.dtype),
            mesh=mesh,
        )
        def gather_kernel(table_hbm, idx_hbm, out_hbm):
            def body(idx_vmem, out_vmem):
                pltpu.sync_copy(table_hbm.at[idx_vmem.at[0]], out_vmem)

            pltpu.emit_pipeline(
                body,
                grid=(N // WINDOW,),
                in_specs=[pl.BlockSpec((1, WINDOW), index_map=lambda i: (0, i))],
                out_specs=[
                    pl.BlockSpec((WINDOW, D_MODEL), index_map=lambda i: (i, 0))
                ],
                core_axis_name=("c", "s"),
                dimension_semantics=(pltpu.PARALLEL,),
            )(idx_hbm, out_hbm)

        return gather_kernel(table, idx)

    out = run(table, idx)
    return out.reshape(B, L, D_MODEL)


# speedup vs baseline: 9.1838x; 1.2381x over previous
import jax
import jax.numpy as jnp
from jax import lax
from jax.experimental import pallas as pl
from jax.experimental.pallas import tpu as pltpu
from jax.experimental.pallas import tpu_sc as plsc

D_MODEL = 128
WINDOW = 128
SUB = 2
CHUNK = SUB * WINDOW
NUM_WORKERS = 32


def kernel(x, table):
    B, L = x.shape
    N = B * L
    rows_per_w = N // NUM_WORKERS
    nchunks = rows_per_w // CHUNK
    idx_rows_per_w = rows_per_w // WINDOW
    idx2d = x.reshape(N // WINDOW, WINDOW)
    mesh = plsc.VectorSubcoreMesh(core_axis_name="c", subcore_axis_name="s")

    @jax.jit
    def run(table, idx2d):
        @pl.kernel(
            out_type=jax.ShapeDtypeStruct((N, D_MODEL), table.dtype),
            mesh=mesh,
            scratch_types=[
                pltpu.VMEM((idx_rows_per_w, WINDOW), jnp.int32),
                pltpu.VMEM((CHUNK, D_MODEL), jnp.float32),
                pltpu.VMEM((CHUNK, D_MODEL), jnp.float32),
                pltpu.SemaphoreType.DMA,
                pltpu.SemaphoreType.DMA,
                pltpu.SemaphoreType.DMA,
                pltpu.SemaphoreType.DMA,
            ],
        )
        def gather_kernel(table_hbm, idx_hbm, out_hbm, idx_v, rows0, rows1,
                          gsem0, gsem1, wsem0, wsem1):
            wid = lax.axis_index("s") * 2 + lax.axis_index("c")
            rbase = wid * rows_per_w
            ibase = wid * idx_rows_per_w

            pltpu.sync_copy(idx_hbm.at[pl.ds(ibase, idx_rows_per_w)], idx_v)

            def gather(c, rows, sem, start):
                for j in range(SUB):
                    cp = pltpu.make_async_copy(
                        table_hbm.at[idx_v.at[c * SUB + j]],
                        rows.at[pl.ds(j * WINDOW, WINDOW)],
                        sem,
                    )
                    cp.start() if start else cp.wait()

            def write(c, rows, sem, start):
                cp = pltpu.make_async_copy(
                    rows, out_hbm.at[pl.ds(rbase + c * CHUNK, CHUNK)], sem
                )
                cp.start() if start else cp.wait()

            bufs = ((rows0, gsem0, wsem0), (rows1, gsem1, wsem1))

            gather(0, rows0, gsem0, True)
            gather(1, rows1, gsem1, True)

            @pl.loop(0, nchunks, step=2)
            def _(i):
                for b, (rows, gsem, wsem) in enumerate(bufs):
                    c = i + b
                    gather(c, rows, gsem, False)
                    write(c, rows, wsem, True)
                    @pl.when(c + 2 < nchunks)
                    def _():
                        write(c, rows, wsem, False)
                        gather(c + 2, rows, gsem, True)

            write(nchunks - 2, rows0, wsem0, False)
            write(nchunks - 1, rows1, wsem1, False)

        return gather_kernel(table, idx2d)

    out = run(table, idx2d)
    return out.reshape(B, L, D_MODEL)
